# baseline (speedup 1.0000x reference)
import functools

import jax
import jax.numpy as jnp
from jax import lax
from jax.experimental import pallas as pl
from jax.experimental.pallas import tpu as pltpu
from jax.experimental.pallas import tpu_sc as plsc

B = 16384
F = 64
N = 1_000_000
L = 16
SLABW = 128
NSLAB = (N + SLABW - 1) // SLABW
TAIL0 = (N // SLABW) * SLABW
TAILS = TAIL0 // SLABW
KMAX = (NSLAB + 15) // 16
SREC = B + KMAX * 16
UROWS = B + 128
NSLOT = 6

_INFO = plsc.get_sparse_core_info()
NC, NS = _INFO.num_cores, _INFO.num_subcores

_mesh = plsc.VectorSubcoreMesh(core_axis_name="c", subcore_axis_name="s")

_f32 = jnp.float32
_i32 = jnp.int32


@functools.partial(
    pl.kernel,
    mesh=_mesh,
    compiler_params=pltpu.CompilerParams(needs_layout_passes=False),
    out_type=(jax.ShapeDtypeStruct((UROWS, SLABW), _f32),
              jax.ShapeDtypeStruct((UROWS, SLABW), _f32)),
    scratch_types=[
        pltpu.VMEM((B,), _i32),
        pltpu.VMEM((1, KMAX * 16), _i32),
        pltpu.VMEM((1, SREC), _i32),
        pltpu.VMEM((NSLOT, F, SLABW), _f32),
        pltpu.VMEM((NSLOT, 1, SLABW), _f32),
        pltpu.VMEM((F, SLABW), _f32),
        pltpu.VMEM((1, F), _i32),
        pltpu.SMEM((KMAX,), _i32),
        pltpu.SemaphoreType.DMA,
        pltpu.SemaphoreType.DMA,
        pltpu.SemaphoreType.DMA,
        pltpu.SemaphoreType.DMA,
        pltpu.SemaphoreType.DMA,
        pltpu.SemaphoreType.DMA,
        pltpu.SemaphoreType.DMA,
    ],
)
def _gather_sc(user_hbm, item_hbm, uft_hbm, ift_hbm, ub_hbm, ib_hbm,
               utail_hbm, itail_hbm, ubtail_hbm, ibtail_hbm,
               uout_hbm, vout_hbm,
               idx_v, hist_v, srec_v, slab_v, bias_v, stage_v, sidx_v,
               occ_s, sem0, sem1, sem2, sem3, sem4, sem5, semo):
    t = lax.axis_index("s")
    core = lax.axis_index("c")
    lane = lax.iota(_i32, L)
    zeros = jnp.zeros((L,), _i32)
    ones = jnp.ones((L,), _i32)
    sems = (sem0, sem1, sem2, sem3, sem4, sem5)

    def side(idx_hbm, tbl_hbm, bias_hbm, tailt_hbm, tailb_hbm, out_hbm):
        pltpu.sync_copy(idx_hbm, idx_v)

        def zclr(j, _):
            hist_v[0, pl.ds(j * 16, 16)] = zeros
            return _
        lax.fori_loop(0, KMAX, zclr, None)

        def hpass(v, _):
            iv = idx_v[pl.ds(v * 16, 16)]
            s = jax.lax.shift_right_logical(iv, 7)
            keep = (s & 15) == t
            addr = jax.lax.shift_right_logical(s, 4) * 16 + lane
            plsc.addupdate_scatter(hist_v, [zeros, addr], ones, mask=keep)
            return _
        lax.fori_loop(0, B // L, hpass, None)

        def ppass(k, carry):
            run, m = carry
            row = hist_v[0, pl.ds(k * 16, 16)]
            rs = jax.lax.reduce_sum_p.bind(row, axes=(0,))
            incl = plsc.cumsum(row)
            hist_v[0, pl.ds(k * 16, 16)] = (incl - row) + run

            @pl.when(rs > 0)
            def _():
                occ_s[m] = (k << 15) | rs

            return (run + ((rs + 15) & ~15),
                    jnp.where(rs > 0, m + 1, m))
        _, M = lax.fori_loop(0, KMAX, ppass, (jnp.int32(0), jnp.int32(0)))

        def spass(v, _):
            iv = idx_v[pl.ds(v * 16, 16)]
            s = jax.lax.shift_right_logical(iv, 7)
            keep = (s & 15) == t
            addr = jax.lax.shift_right_logical(s, 4) * 16 + lane
            rec = ((v * 16 + lane) << 7) | (iv & 127)
            pos = plsc.load_gather(hist_v, [zeros, addr], mask=keep)
            plsc.store_scatter(srec_v, [zeros, pos], rec, mask=keep)
            plsc.addupdate_scatter(hist_v, [zeros, addr], ones, mask=keep)
            return _
        lax.fori_loop(0, B // L, spass, None)

        def dclr(j, _):
            sidx_v[0, pl.ds(j * 16, 16)] = (
                B + ((t * 8 + j * 16 + lane) & 127))
            return _
        lax.fori_loop(0, F // 16, dclr, None)

        def flush_reset(outrow):
            pltpu.async_copy(stage_v, out_hbm.at[sidx_v.at[0]], semo).wait()
            lax.fori_loop(0, F // 16, dclr, None)
            return jnp.int32(0)

        def fetch(m, u):
            pk = occ_s[m]
            k = jax.lax.shift_right_logical(pk, 15)
            s_id = k * 16 + t

            @pl.when(s_id == TAILS)
            def _():
                pltpu.async_copy(tailt_hbm, slab_v.at[u], sems[u])
                pltpu.async_copy(tailb_hbm, bias_v.at[u], sems[u])

            @pl.when(s_id != TAILS)
            def _():
                off = pl.multiple_of(s_id * SLABW, SLABW)
                pltpu.async_copy(tbl_hbm.at[:, pl.ds(off, SLABW)],
                                 slab_v.at[u], sems[u])
                pltpu.async_copy(bias_hbm.at[:, pl.ds(off, SLABW)],
                                 bias_v.at[u], sems[u])

        def extract(m, u, carry):
            outrow, rbase = carry
            pk = occ_s[m]
            cnt = pk & 32767
            pltpu.make_async_copy(uout_hbm.at[pl.ds(0, F)],
                                  slab_v.at[u], sems[u]).wait()
            pltpu.make_async_copy(uout_hbm.at[pl.ds(0, 1)],
                                  bias_v.at[u], sems[u]).wait()

            def gbody(g, carry):
                outrow, _rb = carry
                rec16 = srec_v[0, pl.ds(rbase + g * 16, 16)]
                valid = (g * 16 + lane) < cnt
                outrow = lax.cond(outrow + 16 > F, flush_reset,
                                  lambda r: r, outrow)
                b16 = jax.lax.shift_right_logical(rec16, 7)
                plsc.store_scatter(sidx_v, [zeros, outrow + lane], b16,
                                   mask=valid)
                for tt in range(L):
                    @pl.when(g * 16 + tt < cnt)
                    def _(tt=tt):
                        c = rec16[tt] & 127
                        cs = jnp.full((L,), c, _i32)
                        row = outrow + tt
                        for q in range(F // L):
                            vals = plsc.load_gather(
                                slab_v, [jnp.full((L,), u, _i32),
                                         lane + q * 16, cs])
                            stage_v[row, pl.ds(q * 16, 16)] = vals
                        bv = plsc.load_gather(
                            bias_v, [jnp.full((L,), u, _i32), zeros, cs])
                        stage_v[row, pl.ds(F, 16)] = bv
                return (outrow + jnp.minimum(cnt - g * 16, 16), _rb)

            outrow, _ = lax.fori_loop(0, (cnt + 15) // 16, gbody,
                                      (outrow, rbase))
            return (outrow, rbase + ((cnt + 15) & ~15))

        for j in range(NSLOT - 1):
            @pl.when(j < M)
            def _(j=j):
                fetch(jnp.int32(j), j)

        def mbody(mq, carry):
            for u in range(NSLOT):
                m = mq * NSLOT + u

                def step(carry, m=m, u=u):
                    @pl.when(m + (NSLOT - 1) < M)
                    def _():
                        fetch(m + (NSLOT - 1), (u + NSLOT - 1) % NSLOT)
                    return extract(m, u, carry)

                carry = lax.cond(m < M, step, lambda c: c, carry)
            return carry

        nq = (M + NSLOT - 1) // NSLOT
        outrow, _ = lax.fori_loop(0, nq, mbody,
                                  (jnp.int32(0), jnp.int32(0)))
        lax.cond(outrow > 0, flush_reset, lambda r: r, outrow)

    @pl.when(core == 0)
    def _():
        side(user_hbm, uft_hbm, ub_hbm, utail_hbm, ubtail_hbm, uout_hbm)

    @pl.when(core == 1)
    def _():
        side(item_hbm, ift_hbm, ib_hbm, itail_hbm, ibtail_hbm, vout_hbm)


_BLK = 2048


def _dot_body(u_ref, v_ref, gb_ref, o_ref):
    u = u_ref[...]
    v = v_ref[...]
    prod = u[:, :F] * v[:, :F]
    o_ref[...] = (jnp.sum(prod, axis=1) + u[:, F] + v[:, F]
                  + gb_ref[0, 0])


_dot_tc = pl.pallas_call(
    _dot_body,
    grid=(B // _BLK,),
    in_specs=[
        pl.BlockSpec((_BLK, SLABW), lambda i: (i, 0)),
        pl.BlockSpec((_BLK, SLABW), lambda i: (i, 0)),
        pl.BlockSpec((1, 1), lambda i: (0, 0)),
    ],
    out_specs=pl.BlockSpec((_BLK,), lambda i: (i,)),
    out_shape=jax.ShapeDtypeStruct((B,), _f32),
)


def kernel(user, item, user_factors, item_factors, user_biases, item_biases,
           global_bias):
    uft = user_factors.T
    ift = item_factors.T
    ub = user_biases.T
    ib = item_biases.T
    utail = jnp.pad(uft[:, TAIL0:], ((0, 0), (0, SLABW - (N - TAIL0))))
    itail = jnp.pad(ift[:, TAIL0:], ((0, 0), (0, SLABW - (N - TAIL0))))
    ubtail = jnp.pad(ub[:, TAIL0:], ((0, 0), (0, SLABW - (N - TAIL0))))
    ibtail = jnp.pad(ib[:, TAIL0:], ((0, 0), (0, SLABW - (N - TAIL0))))
    u_st, v_st = _gather_sc(user, item, uft, ift, ub, ib,
                            utail, itail, ubtail, ibtail)
    gb = global_bias.reshape(1, 1).astype(_f32)
    return _dot_tc(u_st, v_st, gb)


# speedup vs baseline: 1.0761x; 1.0761x over previous
import functools

import jax
import jax.numpy as jnp
from jax import lax
from jax.experimental import pallas as pl
from jax.experimental.pallas import tpu as pltpu
from jax.experimental.pallas import tpu_sc as plsc

B = 16384
F = 64
N = 1_000_000
L = 16
SLABW = 128
NSLAB = (N + SLABW - 1) // SLABW
TAIL0 = (N // SLABW) * SLABW
TAILS = TAIL0 // SLABW
KMAX = (NSLAB + 15) // 16
SREC = B + KMAX * 16
UROWS = B + 128
NSLOT = 4

_INFO = plsc.get_sparse_core_info()
NC, NS = _INFO.num_cores, _INFO.num_subcores

_mesh = plsc.VectorSubcoreMesh(core_axis_name="c", subcore_axis_name="s")

_f32 = jnp.float32
_i32 = jnp.int32


@functools.partial(
    pl.kernel,
    mesh=_mesh,
    compiler_params=pltpu.CompilerParams(needs_layout_passes=False),
    out_type=(jax.ShapeDtypeStruct((UROWS, SLABW), _f32),
              jax.ShapeDtypeStruct((UROWS, SLABW), _f32)),
    scratch_types=[
        pltpu.VMEM((B,), _i32),
        pltpu.VMEM((1, KMAX * 16), _i32),
        pltpu.VMEM((1, SREC), _i32),
        pltpu.VMEM((NSLOT, F, SLABW), _f32),
        pltpu.VMEM((NSLOT, 1, SLABW), _f32),
        pltpu.VMEM((F, SLABW), _f32),
        pltpu.VMEM((1, F), _i32),
        pltpu.SMEM((KMAX,), _i32),
        pltpu.SemaphoreType.DMA,
        pltpu.SemaphoreType.DMA,
        pltpu.SemaphoreType.DMA,
        pltpu.SemaphoreType.DMA,
        pltpu.SemaphoreType.DMA,
    ],
)
def _gather_sc(user_hbm, item_hbm, uft_hbm, ift_hbm, ub_hbm, ib_hbm,
               utail_hbm, itail_hbm, ubtail_hbm, ibtail_hbm,
               uout_hbm, vout_hbm,
               idx_v, hist_v, srec_v, slab_v, bias_v, stage_v, sidx_v,
               occ_s, sem0, sem1, sem2, sem3, semo):
    t = lax.axis_index("s")
    core = lax.axis_index("c")
    lane = lax.iota(_i32, L)
    zeros = jnp.zeros((L,), _i32)
    ones = jnp.ones((L,), _i32)
    sems = (sem0, sem1, sem2, sem3)

    def side(idx_hbm, tbl_hbm, bias_hbm, tailt_hbm, tailb_hbm, out_hbm):
        pltpu.sync_copy(idx_hbm, idx_v)

        def zclr(j, _):
            hist_v[0, pl.ds(j * 16, 16)] = zeros
            return _
        lax.fori_loop(0, KMAX, zclr, None)

        def hpass(v, _):
            iv = idx_v[pl.ds(v * 16, 16)]
            s = jax.lax.shift_right_logical(iv, 7)
            keep = (s & 15) == t
            addr = jax.lax.shift_right_logical(s, 4) * 16 + lane
            plsc.addupdate_scatter(hist_v, [zeros, addr], ones, mask=keep)
            return _
        lax.fori_loop(0, B // L, hpass, None)

        def ppass(k, carry):
            run, m = carry
            row = hist_v[0, pl.ds(k * 16, 16)]
            rs = jax.lax.reduce_sum_p.bind(row, axes=(0,))
            incl = plsc.cumsum(row)
            hist_v[0, pl.ds(k * 16, 16)] = (incl - row) + run

            @pl.when(rs > 0)
            def _():
                occ_s[m] = (k << 15) | rs

            return (run + ((rs + 15) & ~15),
                    jnp.where(rs > 0, m + 1, m))
        _, M = lax.fori_loop(0, KMAX, ppass, (jnp.int32(0), jnp.int32(0)))

        def spass(v, _):
            iv = idx_v[pl.ds(v * 16, 16)]
            s = jax.lax.shift_right_logical(iv, 7)
            keep = (s & 15) == t
            addr = jax.lax.shift_right_logical(s, 4) * 16 + lane
            rec = ((v * 16 + lane) << 7) | (iv & 127)
            pos = plsc.load_gather(hist_v, [zeros, addr], mask=keep)
            plsc.store_scatter(srec_v, [zeros, pos], rec, mask=keep)
            plsc.addupdate_scatter(hist_v, [zeros, addr], ones, mask=keep)
            return _
        lax.fori_loop(0, B // L, spass, None)

        def dclr(j, _):
            sidx_v[0, pl.ds(j * 16, 16)] = (
                B + ((t * 8 + j * 16 + lane) & 127))
            return _
        lax.fori_loop(0, F // 16, dclr, None)

        def flush_reset(outrow):
            pltpu.async_copy(stage_v, out_hbm.at[sidx_v.at[0]], semo).wait()
            lax.fori_loop(0, F // 16, dclr, None)
            return jnp.int32(0)

        def fetch(m, u):
            pk = occ_s[m]
            k = jax.lax.shift_right_logical(pk, 15)
            s_id = k * 16 + t

            @pl.when(s_id == TAILS)
            def _():
                pltpu.async_copy(tailt_hbm, slab_v.at[u], sems[u])
                pltpu.async_copy(tailb_hbm, bias_v.at[u], sems[u])

            @pl.when(s_id != TAILS)
            def _():
                off = pl.multiple_of(s_id * SLABW, SLABW)
                pltpu.async_copy(tbl_hbm.at[:, pl.ds(off, SLABW)],
                                 slab_v.at[u], sems[u])
                pltpu.async_copy(bias_hbm.at[:, pl.ds(off, SLABW)],
                                 bias_v.at[u], sems[u])

        def extract(m, u, carry):
            outrow, rbase = carry
            pk = occ_s[m]
            cnt = pk & 32767
            pltpu.make_async_copy(uout_hbm.at[pl.ds(0, F)],
                                  slab_v.at[u], sems[u]).wait()
            pltpu.make_async_copy(uout_hbm.at[pl.ds(0, 1)],
                                  bias_v.at[u], sems[u]).wait()

            def gbody(g, carry):
                outrow, _rb = carry
                rec16 = srec_v[0, pl.ds(rbase + g * 16, 16)]
                valid = (g * 16 + lane) < cnt
                outrow = lax.cond(outrow + 16 > F, flush_reset,
                                  lambda r: r, outrow)
                b16 = jax.lax.shift_right_logical(rec16, 7)
                plsc.store_scatter(sidx_v, [zeros, outrow + lane], b16,
                                   mask=valid)
                for tt in range(L):
                    @pl.when(g * 16 + tt < cnt)
                    def _(tt=tt):
                        c = rec16[tt] & 127
                        cs = jnp.full((L,), c, _i32)
                        row = outrow + tt
                        for q in range(F // L):
                            vals = plsc.load_gather(
                                slab_v, [jnp.full((L,), u, _i32),
                                         lane + q * 16, cs])
                            stage_v[row, pl.ds(q * 16, 16)] = vals
                        bv = plsc.load_gather(
                            bias_v, [jnp.full((L,), u, _i32), zeros, cs])
                        stage_v[row, pl.ds(F, 16)] = bv
                return (outrow + jnp.minimum(cnt - g * 16, 16), _rb)

            outrow, _ = lax.fori_loop(0, (cnt + 15) // 16, gbody,
                                      (outrow, rbase))
            return (outrow, rbase + ((cnt + 15) & ~15))

        for j in range(NSLOT - 1):
            @pl.when(j < M)
            def _(j=j):
                fetch(jnp.int32(j), j)

        def mbody(mq, carry):
            for u in range(NSLOT):
                m = mq * NSLOT + u

                def step(carry, m=m, u=u):
                    @pl.when(m + (NSLOT - 1) < M)
                    def _():
                        fetch(m + (NSLOT - 1), (u + NSLOT - 1) % NSLOT)
                    return extract(m, u, carry)

                carry = lax.cond(m < M, step, lambda c: c, carry)
            return carry

        nq = (M + NSLOT - 1) // NSLOT
        outrow, _ = lax.fori_loop(0, nq, mbody,
                                  (jnp.int32(0), jnp.int32(0)))
        lax.cond(outrow > 0, flush_reset, lambda r: r, outrow)

    @pl.when(core == 0)
    def _():
        side(user_hbm, uft_hbm, ub_hbm, utail_hbm, ubtail_hbm, uout_hbm)

    @pl.when(core == 1)
    def _():
        side(item_hbm, ift_hbm, ib_hbm, itail_hbm, ibtail_hbm, vout_hbm)


_BLK = 2048


def _dot_body(u_ref, v_ref, gb_ref, o_ref):
    u = u_ref[...]
    v = v_ref[...]
    prod = u[:, :F] * v[:, :F]
    o_ref[...] = (jnp.sum(prod, axis=1) + u[:, F] + v[:, F]
                  + gb_ref[0, 0])


_dot_tc = pl.pallas_call(
    _dot_body,
    grid=(B // _BLK,),
    in_specs=[
        pl.BlockSpec((_BLK, SLABW), lambda i: (i, 0)),
        pl.BlockSpec((_BLK, SLABW), lambda i: (i, 0)),
        pl.BlockSpec((1, 1), lambda i: (0, 0)),
    ],
    out_specs=pl.BlockSpec((_BLK,), lambda i: (i,)),
    out_shape=jax.ShapeDtypeStruct((B,), _f32),
)


def kernel(user, item, user_factors, item_factors, user_biases, item_biases,
           global_bias):
    uft = user_factors.T
    ift = item_factors.T
    ub = user_biases.T
    ib = item_biases.T
    utail = jnp.pad(uft[:, TAIL0:], ((0, 0), (0, SLABW - (N - TAIL0))))
    itail = jnp.pad(ift[:, TAIL0:], ((0, 0), (0, SLABW - (N - TAIL0))))
    ubtail = jnp.pad(ub[:, TAIL0:], ((0, 0), (0, SLABW - (N - TAIL0))))
    ibtail = jnp.pad(ib[:, TAIL0:], ((0, 0), (0, SLABW - (N - TAIL0))))
    u_st, v_st = _gather_sc(user, item, uft, ift, ub, ib,
                            utail, itail, ubtail, ibtail)
    gb = global_bias.reshape(1, 1).astype(_f32)
    return _dot_tc(u_st, v_st, gb)
